# Initial kernel scaffold; baseline (speedup 1.0000x reference)
import jax
import jax.numpy as jnp
from jax.experimental import pallas as pl


def kernel(position_ids, embedding_table):
    raise NotImplementedError("write your pallas kernel here")


# speedup vs baseline: 3.5010x; 3.5010x over previous
import functools

import jax
from jax import lax
import jax.numpy as jnp
from jax.experimental import pallas as pl
from jax.experimental.pallas import tpu as pltpu
from jax.experimental.pallas import tpu_sc as plsc

SEQ = 4096
BATCH = 4
HIDDEN = 4096
ROWS = SEQ * BATCH

NW = 32
RPW = ROWS // NW
C = 8
NCH = RPW // C
NBUF = 2

_vector_mesh = plsc.VectorSubcoreMesh(
    core_axis_name="core", subcore_axis_name="subcore"
)


@jax.jit
def _sc_gather(table, indices):
  @functools.partial(
      pl.kernel,
      out_type=jax.ShapeDtypeStruct((ROWS, HIDDEN), table.dtype),
      mesh=_vector_mesh,
      scratch_types=[
          pltpu.VMEM((RPW,), jnp.int32),
          pltpu.VMEM((C, HIDDEN), table.dtype),
          pltpu.VMEM((C, HIDDEN), table.dtype),
          pltpu.SemaphoreType.DMA,
          pltpu.SemaphoreType.DMA,
          pltpu.SemaphoreType.DMA,
          pltpu.SemaphoreType.DMA,
      ],
  )
  def kern(table_hbm, idx_hbm, out_hbm, idx_v, buf0, buf1, g0, g1, o0, o1):
    bufs = (buf0, buf1)
    gsems = (g0, g1)
    osems = (o0, o1)
    wid = lax.axis_index("subcore") * 2 + lax.axis_index("core")
    base = wid * RPW

    pltpu.sync_copy(idx_hbm.at[pl.ds(base, RPW)], idx_v)

    def start_gather(g, b):
      pltpu.async_copy(
          table_hbm.at[idx_v.at[pl.ds(g * C, C)]], bufs[b], gsems[b]
      )

    def wait_gather(b):
      pltpu.make_async_copy(table_hbm.at[pl.ds(0, C)], bufs[b],
                            gsems[b]).wait()

    def wait_out(b):
      pltpu.make_async_copy(bufs[b], out_hbm.at[pl.ds(0, C)],
                            osems[b]).wait()

    for b in range(NBUF):
      start_gather(b, b)

    @pl.loop(0, NCH, step=NBUF)
    def _(c0):
      for b in range(NBUF):
        g = c0 + b
        wait_gather(b)
        pltpu.async_copy(bufs[b], out_hbm.at[pl.ds(base + g * C, C)],
                         osems[b])
        wait_out(b)

        @pl.when(g + NBUF < NCH)
        def _():
          start_gather(g + NBUF, b)

  return kern(table, indices)


def kernel(position_ids, embedding_table):
  idx = jnp.transpose(position_ids).reshape(ROWS).astype(jnp.int32)
  out = _sc_gather(embedding_table, idx)
  return out.reshape(BATCH, SEQ, HIDDEN)
